# baseline (speedup 1.0000x reference)
import functools

import jax
import jax.numpy as jnp
from jax import lax
from jax.experimental import pallas as pl
from jax.experimental.pallas import tpu as pltpu
from jax.experimental.pallas import tpu_sc as plsc

_THRESH = 1.0
_LANES = 16

_NC, _NS = 2, 16
_NW = _NC * _NS


def _make_kernel(B, D, b_per_w):
    n_chunks = D // _LANES
    mesh = plsc.VectorSubcoreMesh(core_axis_name="c", subcore_axis_name="s")

    @functools.partial(
        pl.kernel,
        mesh=mesh,
        out_type=jax.ShapeDtypeStruct((B, D), jnp.float32),
        scratch_types=[
            pltpu.VMEM((b_per_w,), jnp.int32),
            pltpu.VMEM((b_per_w, D), jnp.float32),
            pltpu.VMEM((b_per_w, D), jnp.float32),
            pltpu.SemaphoreType.DMA,
            pltpu.SemaphoreType.DMA,
        ],
        compiler_params=pltpu.CompilerParams(use_tc_tiling_on_sc=False),
    )
    def mask_mul(x_hbm, tab_hbm, idx_hbm, out_hbm, idx_v, rows_v, x_v, g_sem, x_sem):
        wid = lax.axis_index("s") * _NC + lax.axis_index("c")
        base = wid * b_per_w
        pltpu.sync_copy(idx_hbm.at[pl.ds(base, b_per_w)], idx_v)
        gather = pltpu.async_copy(tab_hbm.at[idx_v], rows_v, g_sem)
        xload = pltpu.async_copy(x_hbm.at[pl.ds(base, b_per_w)], x_v, x_sem)
        gather.wait()
        xload.wait()

        def row_body(r, carry):
            for j in range(n_chunks):
                sl = pl.ds(j * _LANES, _LANES)
                e = rows_v[r, sl]
                xv = x_v[r, sl]
                rows_v[r, sl] = jnp.where(e <= _THRESH, xv, 0.0)
            return carry

        lax.fori_loop(0, b_per_w, row_body, 0)
        pltpu.sync_copy(rows_v, out_hbm.at[pl.ds(base, b_per_w)])

    return mask_mul


@jax.jit
def kernel(x, energy_local, batch_idx):
    B, L, H = x.shape
    D = L * H
    x2 = x.reshape(B, D)
    tab = energy_local.reshape(energy_local.shape[0], D)
    idx = batch_idx.astype(jnp.int32)
    out = _make_kernel(B, D, B // _NW)(x2, tab, idx)
    return out.reshape(B, L, H)
